# Initial kernel scaffold; baseline (speedup 1.0000x reference)
import jax
import jax.numpy as jnp
from jax.experimental import pallas as pl


def kernel(indices, in_embeddings):
    raise NotImplementedError("write your pallas kernel here")


# speedup vs baseline: 1.8376x; 1.8376x over previous
import functools

import jax
import jax.numpy as jnp
from jax import lax
from jax.experimental import pallas as pl
from jax.experimental.pallas import tpu as pltpu
from jax.experimental.pallas import tpu_sc as plsc

_CHUNK = 128
_NBUF = 2


@functools.cache
def _build(n_rows, d):
    info = plsc.get_sparse_core_info()
    nc, ns = info.num_cores, info.num_subcores
    nw = nc * ns
    assert n_rows % (nw * _CHUNK) == 0
    n_chunks = n_rows // (nw * _CHUNK)
    mesh = plsc.VectorSubcoreMesh(core_axis_name="c", subcore_axis_name="s")

    @functools.partial(
        pl.kernel,
        mesh=mesh,
        compiler_params=pltpu.CompilerParams(use_tc_tiling_on_sc=False),
        out_type=jax.ShapeDtypeStruct((n_rows, d), jnp.float32),
        scratch_types=[
            pltpu.VMEM((n_chunks, _CHUNK), jnp.int32),
            pltpu.VMEM((_NBUF, _CHUNK, d), jnp.float32),
            pltpu.SemaphoreType.DMA,
            pltpu.SemaphoreType.DMA,
        ],
    )
    def gather_kernel(table_hbm, idx_hbm, out_hbm, idx_v, bufs, sem0, sem1):
        sems = (sem0, sem1)
        wid = lax.axis_index("s") * nc + lax.axis_index("c")
        pltpu.sync_copy(idx_hbm.at[pl.ds(wid * n_chunks, n_chunks)], idx_v)
        row0 = wid * (n_chunks * _CHUNK)
        for b in range(_NBUF):
            pltpu.make_async_copy(
                table_hbm.at[idx_v.at[b]], bufs.at[b], sems[b]).start()

        def step(i, carry):
            g = i * _NBUF
            for b in range(_NBUF):
                c = g + b
                pltpu.make_async_copy(
                    table_hbm.at[idx_v.at[c]], bufs.at[b], sems[b]).wait()
                pltpu.sync_copy(
                    bufs.at[b], out_hbm.at[pl.ds(row0 + c * _CHUNK, _CHUNK)])

                @pl.when(c + _NBUF < n_chunks)
                def _():
                    pltpu.make_async_copy(
                        table_hbm.at[idx_v.at[c + _NBUF]], bufs.at[b],
                        sems[b]).start()
            return carry

        lax.fori_loop(0, n_chunks // _NBUF, step, None)

    return gather_kernel


def kernel(indices, in_embeddings):
    b, h = indices.shape
    _, d = in_embeddings.shape
    n_rows = b * h
    idx = indices.reshape(n_rows // _CHUNK, _CHUNK).astype(jnp.int32)
    out = _build(n_rows, d)(in_embeddings, idx)
    return out.reshape(b, h, d)


# speedup vs baseline: 1.8883x; 1.0276x over previous
import functools

import jax
import jax.numpy as jnp
from jax import lax
from jax.experimental import pallas as pl
from jax.experimental.pallas import tpu as pltpu
from jax.experimental.pallas import tpu_sc as plsc

_CHUNK = 128
_GPF = 4
_NBUF = 2
_FILL = _CHUNK * _GPF


@functools.cache
def _build(n_rows, d):
    info = plsc.get_sparse_core_info()
    nc, ns = info.num_cores, info.num_subcores
    nw = nc * ns
    assert n_rows % (nw * _FILL * _NBUF) == 0
    n_chunks = n_rows // (nw * _CHUNK)
    n_fills = n_chunks // _GPF
    mesh = plsc.VectorSubcoreMesh(core_axis_name="c", subcore_axis_name="s")

    @functools.partial(
        pl.kernel,
        mesh=mesh,
        compiler_params=pltpu.CompilerParams(use_tc_tiling_on_sc=False),
        out_type=jax.ShapeDtypeStruct((n_rows, d), jnp.float32),
        scratch_types=[
            pltpu.VMEM((n_chunks, _CHUNK), jnp.int32),
            pltpu.VMEM((_NBUF, _FILL, d), jnp.float32),
            pltpu.SemaphoreType.DMA,
            pltpu.SemaphoreType.DMA,
        ],
    )
    def gather_kernel(table_hbm, idx_hbm, out_hbm, idx_v, bufs, sem0, sem1):
        sems = (sem0, sem1)
        wid = lax.axis_index("s") * nc + lax.axis_index("c")
        pltpu.sync_copy(idx_hbm.at[pl.ds(wid * n_chunks, n_chunks)], idx_v)
        row0 = wid * (n_chunks * _CHUNK)

        def fire(s, b):
            for j in range(_GPF):
                pltpu.make_async_copy(
                    table_hbm.at[idx_v.at[s * _GPF + j]],
                    bufs.at[b].at[pl.ds(j * _CHUNK, _CHUNK)],
                    sems[b]).start()

        for b in range(_NBUF):
            fire(b, b)

        def step(i, carry):
            g = i * _NBUF
            for b in range(_NBUF):
                s = g + b
                for j in range(_GPF):
                    pltpu.make_async_copy(
                        table_hbm.at[idx_v.at[s * _GPF + j]],
                        bufs.at[b].at[pl.ds(j * _CHUNK, _CHUNK)],
                        sems[b]).wait()
                pltpu.sync_copy(
                    bufs.at[b], out_hbm.at[pl.ds(row0 + s * _FILL, _FILL)])

                @pl.when(s + _NBUF < n_fills)
                def _():
                    fire(s + _NBUF, b)
            return carry

        lax.fori_loop(0, n_fills // _NBUF, step, None)

    return gather_kernel


def kernel(indices, in_embeddings):
    b, h = indices.shape
    _, d = in_embeddings.shape
    n_rows = b * h
    idx = indices.reshape(n_rows // _CHUNK, _CHUNK).astype(jnp.int32)
    out = _build(n_rows, d)(in_embeddings, idx)
    return out.reshape(b, h, d)
